# Initial kernel scaffold; baseline (speedup 1.0000x reference)
import jax
import jax.numpy as jnp
from jax.experimental import pallas as pl


def kernel(x, token_table, pos_table):
    raise NotImplementedError("write your pallas kernel here")


# speedup vs baseline: 2.5499x; 2.5499x over previous
import jax
import jax.numpy as jnp
from jax import lax
from jax.experimental import pallas as pl
from jax.experimental.pallas import tpu as pltpu
from jax.experimental.pallas import tpu_sc as plsc

_NC = 2
_NS = 16
_NW = _NC * _NS
_LANES = 16
_CHUNK = 128


def _make_body(L, D, CH, ROWS_W):
    def body(x_hbm, tok_hbm, pos_hbm, out_hbm,
             idx_v, pos2_v, rows0, rows1, sg0, sg1):
        wid = lax.axis_index("s") * _NC + lax.axis_index("c")
        pltpu.sync_copy(x_hbm.at[wid], idx_v)
        pltpu.sync_copy(pos_hbm, pos2_v.at[pl.ds(0, L)])
        pltpu.sync_copy(pos_hbm, pos2_v.at[pl.ds(L, L)])
        base = wid * ROWS_W

        rows = (rows0, rows1)
        sg = (sg0, sg1)

        pltpu.async_copy(tok_hbm.at[idx_v.at[0]], rows0, sg0)
        pltpu.async_copy(tok_hbm.at[idx_v.at[1]], rows1, sg1)

        @pl.loop(0, CH, step=2)
        def _(i):
            for b in range(2):
                rv = rows[b]
                c = i + b
                pltpu.make_async_copy(tok_hbm.at[idx_v.at[c]], rv, sg[b]).wait()

                p0 = lax.rem(base + c * _CHUNK, L)

                @pl.loop(0, _CHUNK)
                def _(r):
                    for q in range(D // _LANES):
                        cs = pl.ds(q * _LANES, _LANES)
                        rv.at[pl.ds(r, 1), cs][...] = (
                            rv.at[pl.ds(r, 1), cs][...]
                            + pos2_v.at[pl.ds(p0 + r, 1), cs][...]
                        )

                pltpu.sync_copy(rv, out_hbm.at[pl.ds(base + c * _CHUNK, _CHUNK)])

                @pl.when(c + 2 < CH)
                def _():
                    pltpu.async_copy(tok_hbm.at[idx_v.at[c + 2]], rv, sg[b])

    return body


def kernel(x, token_table, pos_table):
    B, L = x.shape
    V, D = token_table.shape
    N = B * L
    ROWS_W = N // _NW
    CH = ROWS_W // _CHUNK

    x_r = x.reshape(_NW, CH, _CHUNK)
    mesh = plsc.VectorSubcoreMesh(core_axis_name="c", subcore_axis_name="s")
    out = pl.kernel(
        _make_body(L, D, CH, ROWS_W),
        out_type=jax.ShapeDtypeStruct((N, D), jnp.float32),
        mesh=mesh,
        compiler_params=pltpu.CompilerParams(use_tc_tiling_on_sc=False),
        scratch_types=[
            pltpu.VMEM((CH, _CHUNK), jnp.int32),
            pltpu.VMEM((2 * L, D), jnp.float32),
            pltpu.VMEM((_CHUNK, D), jnp.float32),
            pltpu.VMEM((_CHUNK, D), jnp.float32),
            pltpu.SemaphoreType.DMA,
            pltpu.SemaphoreType.DMA,
        ],
    )(x_r, token_table, pos_table)
    return out.reshape(B, L, D)


# speedup vs baseline: 3.6288x; 1.4231x over previous
import math

import jax
import jax.numpy as jnp
from jax import lax
from jax.experimental import pallas as pl
from jax.experimental.pallas import tpu as pltpu
from jax.experimental.pallas import tpu_sc as plsc

_NC = 2
_NS = 16
_NW = _NC * _NS
_CHUNK = 128
_NBUF = 8


def _make_body(CH, ROWS_W, PERIOD):
    def body(x_hbm, tok_hbm, pose_hbm, out_hbm, idx_v, rv, si, sg, sw):
        wid = lax.axis_index("s") * _NC + lax.axis_index("c")
        pltpu.sync_copy(x_hbm.at[wid], idx_v)
        base = wid * ROWS_W

        def init(c, b):
            pltpu.async_copy(pose_hbm.at[lax.rem(c, PERIOD)], rv.at[b],
                             si.at[b])

        def init_wait(c, b):
            pltpu.make_async_copy(pose_hbm.at[lax.rem(c, PERIOD)], rv.at[b],
                                  si.at[b]).wait()

        def gather(c, b):
            pltpu.async_copy(tok_hbm.at[idx_v.at[c]], rv.at[b], sg.at[b],
                             add=True)

        def gather_wait(c, b):
            pltpu.make_async_copy(
                tok_hbm.at[idx_v.at[c]], rv.at[b], sg.at[b]).wait()

        def write(c, b):
            pltpu.async_copy(
                rv.at[b], out_hbm.at[pl.ds(base + c * _CHUNK, _CHUNK)],
                sw.at[b])

        def write_wait(c, b):
            pltpu.make_async_copy(
                rv.at[b], out_hbm.at[pl.ds(base + c * _CHUNK, _CHUNK)],
                sw.at[b]).wait()

        for c0 in range(4):
            init(c0, c0)
        init_wait(0, 0)
        gather(0, 0)
        init_wait(1, 1)
        gather(1, 1)

        @pl.loop(0, CH, step=_NBUF)
        def _(t):
            for k in range(_NBUF):
                c = t + k
                b2 = (k + 2) % _NBUF
                b4 = (k + 4) % _NBUF

                @pl.when(c + 4 < CH)
                def _():
                    @pl.when(c >= 4)
                    def _():
                        write_wait(c - 4, b4)

                    init(c + 4, b4)

                @pl.when(c + 2 < CH)
                def _():
                    init_wait(c + 2, b2)
                    gather(c + 2, b2)

                gather_wait(c, k)
                write(c, k)

        for k in range(_NBUF):
            write_wait(CH - _NBUF + k, k)

    return body


def kernel(x, token_table, pos_table):
    B, L = x.shape
    V, D = token_table.shape
    N = B * L
    ROWS_W = N // _NW
    CH = ROWS_W // _CHUNK
    PERIOD = math.lcm(_CHUNK, L) // _CHUNK

    x_r = x.reshape(_NW, CH, _CHUNK)
    reps = PERIOD * _CHUNK // L
    pos_exp = jnp.tile(pos_table, (reps, 1)).reshape(PERIOD, _CHUNK, D)

    mesh = plsc.VectorSubcoreMesh(core_axis_name="c", subcore_axis_name="s")
    out = pl.kernel(
        _make_body(CH, ROWS_W, PERIOD),
        out_type=jax.ShapeDtypeStruct((N, D), jnp.float32),
        mesh=mesh,
        compiler_params=pltpu.CompilerParams(use_tc_tiling_on_sc=False),
        scratch_types=[
            pltpu.VMEM((CH, _CHUNK), jnp.int32),
            pltpu.VMEM((_NBUF, _CHUNK, D), jnp.float32),
            pltpu.SemaphoreType.DMA((_NBUF,)),
            pltpu.SemaphoreType.DMA((_NBUF,)),
            pltpu.SemaphoreType.DMA((_NBUF,)),
        ],
    )(x_r, token_table, pos_exp)
    return out.reshape(B, L, D)
